# baseline (speedup 1.0000x reference)
import functools

import jax
import jax.numpy as jnp
from jax import lax
from jax.experimental import pallas as pl
from jax.experimental.pallas import tpu as pltpu
from jax.experimental.pallas import tpu_sc as plsc

N_NODES = 10000
EMB = 256
HID = 512
N_EDGES = 160000

NC = 2
NS = 16
NW = NC * NS
CH = 64
EW = 5120
NCH = EW // CH
E_PAD = EW * NW
LANES = 16
KB = HID // (2 * LANES)


def _mm_body(x_ref, w_ref, b_ref, o_ref):
    o_ref[...] = (
        jnp.dot(x_ref[...], w_ref[...], preferred_element_type=jnp.float32)
        + 0.5 * b_ref[...]
    ).astype(jnp.bfloat16)


def _node_transform(node_feat, W1, b1):
    return pl.pallas_call(
        _mm_body,
        grid=(5,),
        in_specs=[
            pl.BlockSpec((N_NODES // 5, EMB), lambda i: (i, 0)),
            pl.BlockSpec((EMB, HID), lambda i: (0, 0)),
            pl.BlockSpec((1, HID), lambda i: (0, 0)),
        ],
        out_specs=pl.BlockSpec((N_NODES // 5, HID), lambda i: (i, 0)),
        out_shape=jax.ShapeDtypeStruct((N_NODES, HID), jnp.bfloat16),
    )(node_feat, W1, b1.reshape(1, HID))


def _edge_body(z_hbm, src_hbm, dst_hbm, w2e_hbm, w2o_hbm, b2_hbm, out_hbm,
               src_v, dst_v, zi, zj, w2e_v, w2o_v, b2_v, out_v, sem0, sem1):
    wid = lax.axis_index("s") * NC + lax.axis_index("c")
    base = wid * EW
    pltpu.sync_copy(src_hbm.at[pl.ds(base, EW)], src_v)
    pltpu.sync_copy(dst_hbm.at[pl.ds(base, EW)], dst_v)
    pltpu.sync_copy(w2e_hbm, w2e_v)
    pltpu.sync_copy(w2o_hbm, w2o_v)
    pltpu.sync_copy(b2_hbm, b2_v)

    b2vec = b2_v[...]
    w2e_r = [w2e_v[pl.ds(k * LANES, LANES)] for k in range(KB)]
    w2o_r = [w2o_v[pl.ds(k * LANES, LANES)] for k in range(KB)]
    lanes = lax.iota(jnp.int32, LANES)
    sems = (sem0, sem1)

    def start_chunk(c, slot):
        off = pl.multiple_of(c * CH, CH)
        pltpu.async_copy(z_hbm.at[src_v.at[pl.ds(off, CH)]],
                         zi.at[slot], sems[slot])
        pltpu.async_copy(z_hbm.at[dst_v.at[pl.ds(off, CH)]],
                         zj.at[slot], sems[slot])

    def wait_chunk(c, slot):
        off = pl.multiple_of(c * CH, CH)
        pltpu.make_async_copy(z_hbm.at[src_v.at[pl.ds(off, CH)]],
                              zi.at[slot], sems[slot]).wait()
        pltpu.make_async_copy(z_hbm.at[dst_v.at[pl.ds(off, CH)]],
                              zj.at[slot], sems[slot]).wait()

    start_chunk(0, 0)
    start_chunk(1, 1)

    def pair_body(i, carry):
        for slot in range(2):
            c = i * 2 + slot
            wait_chunk(c, slot)
            off = pl.multiple_of(c * CH, CH)
            for g in range(CH // LANES):
                def edge_pair_body(j, res):
                    def one_edge(row):
                        ae = jnp.zeros((LANES,), jnp.float32)
                        ao = jnp.zeros((LANES,), jnp.float32)
                        for k in range(KB):
                            sl = pl.ds(k * 2 * LANES, 2 * LANES)
                            hb = jnp.maximum(
                                zi[slot, row, sl] + zj[slot, row, sl],
                                jnp.bfloat16(0.0))
                            he, ho = plsc.unpack(
                                hb, format=plsc.PackFormat.INTERLEAVED,
                                preferred_element_type=jnp.float32)
                            ae = ae + he * w2e_r[k]
                            ao = ao + ho * w2o_r[k]
                        return jnp.sum(ae + ao)
                    row0 = g * LANES + 2 * j
                    s0 = one_edge(row0)
                    s1 = one_edge(row0 + 1)
                    res = jnp.where(lanes == 2 * j, s0, res)
                    return jnp.where(lanes == 2 * j + 1, s1, res)
                res = lax.fori_loop(0, LANES // 2, edge_pair_body, b2vec)
                out_v[pl.ds(off + g * LANES, LANES)] = res
            @pl.when(c + 2 < NCH)
            def _():
                start_chunk(c + 2, slot)
        return carry

    lax.fori_loop(0, NCH // 2, pair_body, 0)
    pltpu.sync_copy(out_v, out_hbm.at[pl.ds(base, EW)])


_edge_kernel = functools.partial(
    pl.kernel,
    out_type=jax.ShapeDtypeStruct((E_PAD,), jnp.float32),
    mesh=plsc.VectorSubcoreMesh(core_axis_name="c", subcore_axis_name="s"),
    compiler_params=pltpu.CompilerParams(
        use_tc_tiling_on_sc=False, needs_layout_passes=False),
    scratch_types=[
        pltpu.VMEM((EW,), jnp.int32),
        pltpu.VMEM((EW,), jnp.int32),
        pltpu.VMEM((2, CH, HID), jnp.bfloat16),
        pltpu.VMEM((2, CH, HID), jnp.bfloat16),
        pltpu.VMEM((HID // 2,), jnp.float32),
        pltpu.VMEM((HID // 2,), jnp.float32),
        pltpu.VMEM((LANES,), jnp.float32),
        pltpu.VMEM((EW,), jnp.float32),
        pltpu.SemaphoreType.DMA,
        pltpu.SemaphoreType.DMA,
    ],
)(_edge_body)


def kernel(node_feat, edge_index, W1, b1, W2, b2):
    z = _node_transform(node_feat, W1, b1)
    pad = E_PAD - N_EDGES
    src = jnp.concatenate(
        [edge_index[0].astype(jnp.int32), jnp.zeros((pad,), jnp.int32)])
    dst = jnp.concatenate(
        [edge_index[1].astype(jnp.int32), jnp.zeros((pad,), jnp.int32)])
    w2_pairs = W2.reshape(HID // 2, 2)
    w2_even = w2_pairs[:, 0]
    w2_odd = w2_pairs[:, 1]
    b2_vec = jnp.broadcast_to(b2, (LANES,))
    out = _edge_kernel(z, src, dst, w2_even, w2_odd, b2_vec)
    return out[:N_EDGES]


# speedup vs baseline: 1.0107x; 1.0107x over previous
import functools

import jax
import jax.numpy as jnp
from jax import lax
from jax.experimental import pallas as pl
from jax.experimental.pallas import tpu as pltpu
from jax.experimental.pallas import tpu_sc as plsc

N_NODES = 10000
EMB = 256
HID = 512
N_EDGES = 160000

NC = 2
NS = 16
NW = NC * NS
CH = 64
EW = 5120
NCH = EW // CH
E_PAD = EW * NW
LANES = 16
KB = HID // (2 * LANES)


def _mm_body(x_ref, w_ref, b_ref, o_ref):
    o_ref[...] = (
        jnp.dot(x_ref[...], w_ref[...], preferred_element_type=jnp.float32)
        + 0.5 * b_ref[...]
    ).astype(jnp.bfloat16)


def _node_transform(node_feat, W1, b1):
    return pl.pallas_call(
        _mm_body,
        grid=(5,),
        in_specs=[
            pl.BlockSpec((N_NODES // 5, EMB), lambda i: (i, 0)),
            pl.BlockSpec((EMB, HID), lambda i: (0, 0)),
            pl.BlockSpec((1, HID), lambda i: (0, 0)),
        ],
        out_specs=pl.BlockSpec((N_NODES // 5, HID), lambda i: (i, 0)),
        out_shape=jax.ShapeDtypeStruct((N_NODES, HID), jnp.bfloat16),
    )(node_feat, W1, b1.reshape(1, HID))


def _edge_body(z_hbm, src_hbm, dst_hbm, w2e_hbm, w2o_hbm, b2_hbm, out_hbm,
               src_v, dst_v, zi, zj, w2e_v, w2o_v, b2_v, out_v, sem0, sem1):
    wid = lax.axis_index("s") * NC + lax.axis_index("c")
    base = wid * EW
    pltpu.sync_copy(src_hbm.at[pl.ds(base, EW)], src_v)
    pltpu.sync_copy(dst_hbm.at[pl.ds(base, EW)], dst_v)
    pltpu.sync_copy(w2e_hbm, w2e_v)
    pltpu.sync_copy(w2o_hbm, w2o_v)
    pltpu.sync_copy(b2_hbm, b2_v)

    b2vec = b2_v[...]
    w2e_r = [w2e_v[pl.ds(k * LANES, LANES)] for k in range(KB)]
    w2o_r = [w2o_v[pl.ds(k * LANES, LANES)] for k in range(KB)]
    lanes = lax.iota(jnp.int32, LANES)
    sems = (sem0, sem1)

    def start_chunk(c, slot):
        off = pl.multiple_of(c * CH, CH)
        pltpu.async_copy(z_hbm.at[src_v.at[pl.ds(off, CH)]],
                         zi.at[slot], sems[slot])
        pltpu.async_copy(z_hbm.at[dst_v.at[pl.ds(off, CH)]],
                         zj.at[slot], sems[slot])

    def wait_chunk(c, slot):
        off = pl.multiple_of(c * CH, CH)
        pltpu.make_async_copy(z_hbm.at[src_v.at[pl.ds(off, CH)]],
                              zi.at[slot], sems[slot]).wait()
        pltpu.make_async_copy(z_hbm.at[dst_v.at[pl.ds(off, CH)]],
                              zj.at[slot], sems[slot]).wait()

    start_chunk(0, 0)
    start_chunk(1, 1)

    def pair_body(i, carry):
        for slot in range(2):
            c = i * 2 + slot
            wait_chunk(c, slot)
            off = pl.multiple_of(c * CH, CH)
            for g in range(CH // LANES):
                def edge_body(e, res):
                    row = g * LANES + e
                    ae = jnp.zeros((LANES,), jnp.float32)
                    ao = jnp.zeros((LANES,), jnp.float32)
                    for k in range(KB):
                        sl = pl.ds(k * 2 * LANES, 2 * LANES)
                        hb = jnp.maximum(
                            zi[slot, row, sl] + zj[slot, row, sl],
                            jnp.bfloat16(0.0))
                        he, ho = plsc.unpack(
                            hb, format=plsc.PackFormat.INTERLEAVED,
                            preferred_element_type=jnp.float32)
                        ae = ae + he * w2e_r[k]
                        ao = ao + ho * w2o_r[k]
                    s = jnp.sum(ae + ao)
                    return jnp.where(lanes == e, s, res)
                res = lax.fori_loop(0, LANES, edge_body, b2vec)
                out_v[pl.ds(off + g * LANES, LANES)] = res
            @pl.when(c + 2 < NCH)
            def _():
                start_chunk(c + 2, slot)
        return carry

    lax.fori_loop(0, NCH // 2, pair_body, 0)
    pltpu.sync_copy(out_v, out_hbm.at[pl.ds(base, EW)])


_edge_kernel = functools.partial(
    pl.kernel,
    out_type=jax.ShapeDtypeStruct((E_PAD,), jnp.float32),
    mesh=plsc.VectorSubcoreMesh(core_axis_name="c", subcore_axis_name="s"),
    compiler_params=pltpu.CompilerParams(
        use_tc_tiling_on_sc=False, needs_layout_passes=False),
    scratch_types=[
        pltpu.VMEM((EW,), jnp.int32),
        pltpu.VMEM((EW,), jnp.int32),
        pltpu.VMEM((2, CH, HID), jnp.bfloat16),
        pltpu.VMEM((2, CH, HID), jnp.bfloat16),
        pltpu.VMEM((HID // 2,), jnp.float32),
        pltpu.VMEM((HID // 2,), jnp.float32),
        pltpu.VMEM((LANES,), jnp.float32),
        pltpu.VMEM((EW,), jnp.float32),
        pltpu.SemaphoreType.DMA,
        pltpu.SemaphoreType.DMA,
    ],
)(_edge_body)


def kernel(node_feat, edge_index, W1, b1, W2, b2):
    z = _node_transform(node_feat, W1, b1)
    pad = E_PAD - N_EDGES
    src = jnp.concatenate(
        [edge_index[0].astype(jnp.int32), jnp.zeros((pad,), jnp.int32)])
    dst = jnp.concatenate(
        [edge_index[1].astype(jnp.int32), jnp.zeros((pad,), jnp.int32)])
    w2_pairs = W2.reshape(HID // 2, 2)
    w2_even = w2_pairs[:, 0]
    w2_odd = w2_pairs[:, 1]
    b2_vec = jnp.broadcast_to(b2, (LANES,))
    out = _edge_kernel(z, src, dst, w2_even, w2_odd, b2_vec)
    return out[:N_EDGES]


# speedup vs baseline: 1.4807x; 1.4650x over previous
import functools

import jax
import jax.numpy as jnp
from jax import lax
from jax.experimental import pallas as pl
from jax.experimental.pallas import tpu as pltpu
from jax.experimental.pallas import tpu_sc as plsc

N_NODES = 10000
EMB = 256
HID = 512
N_EDGES = 160000

NC = 2
NS = 16
NW = NC * NS
CH = 32
EW = 5056
NCH = EW // CH
E_PAD = EW * NW
LANES = 16
KB = HID // (2 * LANES)


def _mm_body(x_ref, w_ref, b_ref, o_ref):
    o_ref[...] = (
        jnp.dot(x_ref[...], w_ref[...], preferred_element_type=jnp.float32)
        + 0.5 * b_ref[...]
    ).astype(jnp.bfloat16)


def _node_transform(node_feat, W1, b1):
    return pl.pallas_call(
        _mm_body,
        grid=(5,),
        in_specs=[
            pl.BlockSpec((N_NODES // 5, EMB), lambda i: (i, 0)),
            pl.BlockSpec((EMB, HID), lambda i: (0, 0)),
            pl.BlockSpec((1, HID), lambda i: (0, 0)),
        ],
        out_specs=pl.BlockSpec((N_NODES // 5, HID), lambda i: (i, 0)),
        out_shape=jax.ShapeDtypeStruct((N_NODES, HID), jnp.bfloat16),
    )(node_feat, W1, b1.reshape(1, HID))


def _edge_body(z_hbm, src_hbm, dst_hbm, w2e_hbm, w2o_hbm, b2_hbm, out_hbm,
               src_v, dst_v, zi, zj, w2e_v, w2o_v, b2_v, out_v, sem0, sem1):
    wid = lax.axis_index("s") * NC + lax.axis_index("c")
    base = wid * EW
    pltpu.sync_copy(src_hbm.at[pl.ds(base, EW)], src_v)
    pltpu.sync_copy(dst_hbm.at[pl.ds(base, EW)], dst_v)
    pltpu.sync_copy(w2e_hbm, w2e_v)
    pltpu.sync_copy(w2o_hbm, w2o_v)
    pltpu.sync_copy(b2_hbm, b2_v)

    b2vec = b2_v[...]
    w2e_r = [w2e_v[pl.ds(k * LANES, LANES)] for k in range(KB)]
    w2o_r = [w2o_v[pl.ds(k * LANES, LANES)] for k in range(KB)]
    lanes = lax.iota(jnp.int32, LANES)
    sems = (sem0, sem1)

    def start_chunk(c, slot):
        off = pl.multiple_of(c * CH, CH)
        pltpu.async_copy(z_hbm.at[src_v.at[pl.ds(off, CH)]],
                         zi.at[slot], sems[slot])
        pltpu.async_copy(z_hbm.at[dst_v.at[pl.ds(off, CH)]],
                         zj.at[slot], sems[slot])

    def wait_chunk(c, slot):
        off = pl.multiple_of(c * CH, CH)
        pltpu.make_async_copy(z_hbm.at[src_v.at[pl.ds(off, CH)]],
                              zi.at[slot], sems[slot]).wait()
        pltpu.make_async_copy(z_hbm.at[dst_v.at[pl.ds(off, CH)]],
                              zj.at[slot], sems[slot]).wait()

    start_chunk(0, 0)
    start_chunk(1, 1)

    def pair_body(i, carry):
        for slot in range(2):
            c = i * 2 + slot
            wait_chunk(c, slot)
            off = pl.multiple_of(c * CH, CH)
            for g in range(CH // LANES):
                def edge_body(e, res):
                    row = g * LANES + e
                    ae = jnp.zeros((LANES,), jnp.float32)
                    ao = jnp.zeros((LANES,), jnp.float32)
                    for k in range(KB):
                        sl = pl.ds(k * 2 * LANES, 2 * LANES)
                        hb = jnp.maximum(
                            zi[slot, row, sl] + zj[slot, row, sl],
                            jnp.bfloat16(0.0))
                        he, ho = plsc.unpack(
                            hb, format=plsc.PackFormat.INTERLEAVED,
                            preferred_element_type=jnp.float32)
                        ae = ae + he * w2e_r[k]
                        ao = ao + ho * w2o_r[k]
                    s = jnp.sum(ae + ao)
                    return jnp.where(lanes == e, s, res)
                res = lax.fori_loop(0, LANES, edge_body, b2vec)
                out_v[pl.ds(off + g * LANES, LANES)] = res
            @pl.when(c + 2 < NCH)
            def _():
                start_chunk(c + 2, slot)
        return carry

    lax.fori_loop(0, NCH // 2, pair_body, 0)
    pltpu.sync_copy(out_v, out_hbm.at[pl.ds(base, EW)])


_edge_kernel = functools.partial(
    pl.kernel,
    out_type=jax.ShapeDtypeStruct((E_PAD,), jnp.float32),
    mesh=plsc.VectorSubcoreMesh(core_axis_name="c", subcore_axis_name="s"),
    compiler_params=pltpu.CompilerParams(
        use_tc_tiling_on_sc=False, needs_layout_passes=False),
    scratch_types=[
        pltpu.VMEM((EW,), jnp.int32),
        pltpu.VMEM((EW,), jnp.int32),
        pltpu.VMEM((2, CH, HID), jnp.bfloat16),
        pltpu.VMEM((2, CH, HID), jnp.bfloat16),
        pltpu.VMEM((HID // 2,), jnp.float32),
        pltpu.VMEM((HID // 2,), jnp.float32),
        pltpu.VMEM((LANES,), jnp.float32),
        pltpu.VMEM((EW,), jnp.float32),
        pltpu.SemaphoreType.DMA,
        pltpu.SemaphoreType.DMA,
    ],
)(_edge_body)


def kernel(node_feat, edge_index, W1, b1, W2, b2):
    z = _node_transform(node_feat, W1, b1)
    pad = E_PAD - N_EDGES
    src = jnp.concatenate(
        [edge_index[0].astype(jnp.int32), jnp.zeros((pad,), jnp.int32)])
    dst = jnp.concatenate(
        [edge_index[1].astype(jnp.int32), jnp.zeros((pad,), jnp.int32)])
    w2_pairs = W2.reshape(HID // 2, 2)
    w2_even = w2_pairs[:, 0]
    w2_odd = w2_pairs[:, 1]
    b2_vec = jnp.broadcast_to(b2, (LANES,))
    out = _edge_kernel(z, src, dst, w2_even, w2_odd, b2_vec)
    return out[:N_EDGES]
